# Initial kernel scaffold; baseline (speedup 1.0000x reference)
import jax
import jax.numpy as jnp
from jax.experimental import pallas as pl


def kernel(input, adj, edge_weight, W, b):
    raise NotImplementedError("write your pallas kernel here")


# speedup vs baseline: 4.7382x; 4.7382x over previous
import functools

import jax
import jax.numpy as jnp
from jax import lax
from jax.experimental import pallas as pl
from jax.experimental.pallas import tpu as pltpu
from jax.experimental.pallas import tpu_sc as plsc

NSC = 2
TPS = 16
NT = NSC * TPS
K = 128
LANES = 16


@functools.partial(jax.jit, static_argnums=(4, 5, 6))
def _sc_aggregate(x, colv, rowv, eww, N, D, NCH):
    rpt = 8 * (-(-N // (TPS * 8)))
    NP = TPS * rpt

    mesh = plsc.VectorSubcoreMesh(core_axis_name="c", subcore_axis_name="s")

    @functools.partial(
        pl.kernel,
        mesh=mesh,
        out_type=jax.ShapeDtypeStruct((NSC, NP, D), jnp.float32),
        scratch_types=[
            pltpu.VMEM((NCH, K), jnp.int32),
            pltpu.VMEM((NCH, K), jnp.int32),
            pltpu.VMEM((NCH, K), jnp.float32),
            pltpu.VMEM((K, D), jnp.float32),
            pltpu.VMEM_SHARED((NP, D), jnp.float32),
            pltpu.SemaphoreType.DMA,
        ],
    )
    def sc(x_hbm, col_hbm, row_hbm, ew_hbm, out_hbm,
           colr, rowr, ewr, rbuf, acc, sem):
        cid = lax.axis_index("c")
        sid = lax.axis_index("s")
        tid = cid * TPS + sid

        pltpu.sync_copy(col_hbm.at[tid], colr)
        pltpu.sync_copy(row_hbm.at[tid], rowr)
        pltpu.sync_copy(ew_hbm.at[tid], ewr)

        def zrow(r, carry):
            for c in range(D // LANES):
                rbuf[r, pl.ds(c * LANES, LANES)] = jnp.zeros(
                    (LANES,), jnp.float32)
            return carry
        lax.fori_loop(0, K, zrow, 0)

        zbase = sid * rpt
        nfull = rpt // K
        rem = rpt - nfull * K

        def zcp(i, carry):
            pltpu.sync_copy(rbuf, acc.at[pl.ds(zbase + i * K, K)])
            return carry
        lax.fori_loop(0, nfull, zcp, 0)
        if rem:
            pltpu.sync_copy(rbuf.at[pl.ds(0, rem)],
                            acc.at[pl.ds(zbase + nfull * K, rem)])
        plsc.subcore_barrier()

        def chunk(j, carry):
            pltpu.async_copy(x_hbm.at[colr.at[j]], rbuf, sem).wait()

            def rowblk(g, c2):
                wv = ewr[j, pl.ds(g * LANES, LANES)]
                for u in range(LANES):
                    w = wv[u]
                    r = g * LANES + u
                    for c in range(D // LANES):
                        sl = pl.ds(c * LANES, LANES)
                        rbuf[r, sl] = rbuf[r, sl] * w
                return c2
            lax.fori_loop(0, K // LANES, rowblk, 0)

            pltpu.sync_copy(rbuf, acc.at[rowr.at[j]], add=True)
            return carry
        lax.fori_loop(0, NCH, chunk, 0)

        plsc.subcore_barrier()
        pltpu.sync_copy(acc.at[pl.ds(zbase, rpt)],
                        out_hbm.at[cid, pl.ds(zbase, rpt)])

    return sc(x, colv, rowv, eww)


def _tc_combine_matmul(P, W, b, N):
    _, _, D = P.shape
    DO = W.shape[1]
    BM = 1000

    def body(p_ref, w_ref, b_ref, o_ref):
        s = p_ref[0] + p_ref[1]
        o_ref[...] = (
            jnp.dot(s, w_ref[...], preferred_element_type=jnp.float32)
            + b_ref[...]
        )

    return pl.pallas_call(
        body,
        grid=(N // BM,),
        in_specs=[
            pl.BlockSpec((NSC, BM, D), lambda i: (0, i, 0)),
            pl.BlockSpec((D, DO), lambda i: (0, 0)),
            pl.BlockSpec((1, DO), lambda i: (0, 0)),
        ],
        out_specs=pl.BlockSpec((BM, DO), lambda i: (i, 0)),
        out_shape=jax.ShapeDtypeStruct((N, DO), jnp.float32),
    )(P, W, b.reshape(1, DO))


def kernel(input, adj, edge_weight, W, b):
    x = input
    N, D = x.shape
    E = edge_weight.shape[0]

    ept = -(-E // NT)
    NCH = -(-ept // K)
    EPT = NCH * K

    col = adj[1]
    row = adj[0]
    if E % NT:
        pad0 = NT * ept - E
        col = jnp.pad(col, (0, pad0))
        row = jnp.pad(row, (0, pad0))
        ew = jnp.pad(edge_weight, (0, pad0))
    else:
        ew = edge_weight
    colv = jnp.pad(col.reshape(NT, ept), ((0, 0), (0, EPT - ept)))
    rowv = jnp.pad(row.reshape(NT, ept), ((0, 0), (0, EPT - ept)))
    eww = jnp.pad(ew.reshape(NT, ept), ((0, 0), (0, EPT - ept)))

    P = _sc_aggregate(
        x,
        colv.reshape(NT, NCH, K),
        rowv.reshape(NT, NCH, K),
        eww.reshape(NT, NCH, K),
        N, D, NCH,
    )
    return _tc_combine_matmul(P, W, b, N)
